# Initial kernel scaffold; baseline (speedup 1.0000x reference)
import jax
import jax.numpy as jnp
from jax.experimental import pallas as pl


def kernel(base_fs, ws, vb_index, vv_index):
    raise NotImplementedError("write your pallas kernel here")


# speedup vs baseline: 85.0218x; 85.0218x over previous
import functools

import jax
import jax.numpy as jnp
from jax import lax
from jax.experimental import pallas as pl
from jax.experimental.pallas import tpu as pltpu
from jax.experimental.pallas import tpu_sc as plsc

BASE_NUM = 64
VNUM = 50000
DIM = 9
K = 8
B = 16
ROW = B * DIM
VB = 400
NBLK = VNUM // VB
NWORKERS = 32
OBN = VB * DIM


def _sc_body(t_hbm, ws_hbm, vb_hbm, out_hbm, tbl, wv, ib, ob, sem):
    c = lax.axis_index("c")
    s = lax.axis_index("s")
    wid = s * 2 + c

    pltpu.sync_copy(t_hbm, tbl)

    lane = lax.broadcasted_iota(jnp.int32, (16,), 0)
    obase = lane * OBN
    gbase = lane * K

    def blk_body(i, _):
        blk = wid + i * NWORKERS
        v0 = blk * VB
        pltpu.sync_copy(ws_hbm.at[pl.ds(v0 * K, VB * K)], wv)
        pltpu.sync_copy(vb_hbm.at[pl.ds(v0 * K, VB * K)], ib)

        def sm_body(g, _):
            idx0 = gbase + g * (16 * K)
            w = [plsc.load_gather(wv, [idx0 + k]) for k in range(K)]
            m = w[0]
            for k in range(1, K):
                m = jnp.maximum(m, w[k])
            e = [jnp.exp(wk - m) for wk in w]
            ssum = e[0]
            for k in range(1, K):
                ssum = ssum + e[k]
            inv = 1.0 / (ssum + 1e-16)
            for k in range(K):
                plsc.store_scatter(wv, [idx0 + k], e[k] * inv)
            return 0

        lax.fori_loop(0, VB // 16, sm_body, 0, unroll=2)

        def v_body(v2, _):
            e0 = v2 * 16
            wvec = wv[pl.ds(e0, 16)]
            ivec = ib[pl.ds(e0, 16)]
            for half in range(2):
                accs = None
                for k in range(K):
                    j = half * K + k
                    off = ivec[j] * ROW
                    wk = wvec[j]
                    rows = [
                        tbl[pl.ds(off + d * 16, 16)] * wk for d in range(DIM)
                    ]
                    if accs is None:
                        accs = rows
                    else:
                        accs = [a + r for a, r in zip(accs, rows)]
                sbase = obase + (v2 * 2 + half) * DIM
                for d in range(DIM):
                    plsc.store_scatter(ob, [sbase + d], accs[d])
            return 0

        lax.fori_loop(0, VB // 2, v_body, 0)

        cps = [
            pltpu.async_copy(
                ob.at[pl.ds(b * OBN, OBN)],
                out_hbm.at[pl.ds(b * (VNUM * DIM) + v0 * DIM, OBN)],
                sem,
            )
            for b in range(B)
        ]
        for cp in cps:
            cp.wait()
        return 0

    nblk = (NBLK - wid + NWORKERS - 1) // NWORKERS
    lax.fori_loop(0, nblk, blk_body, 0)


@jax.jit
def _skin(t2, ws, vb):
    run = pl.kernel(
        _sc_body,
        out_type=jax.ShapeDtypeStruct((B * VNUM * DIM,), jnp.float32),
        mesh=plsc.VectorSubcoreMesh(core_axis_name="c", subcore_axis_name="s"),
        scratch_types=[
            pltpu.VMEM((BASE_NUM * ROW,), jnp.float32),
            pltpu.VMEM((VB * K,), jnp.float32),
            pltpu.VMEM((VB * K,), jnp.int32),
            pltpu.VMEM((B * OBN,), jnp.float32),
            pltpu.SemaphoreType.DMA,
        ],
        compiler_params=pltpu.CompilerParams(needs_layout_passes=False),
    )
    return run(t2, ws, vb)


def kernel(base_fs, ws, vb_index, vv_index):
    t2 = (
        base_fs.reshape(B, BASE_NUM, DIM)
        .transpose(1, 2, 0)
        .reshape(BASE_NUM * ROW)
    )
    out = _skin(t2, ws, vb_index)
    return out.reshape(B, VNUM, DIM)
